# Initial kernel scaffold; baseline (speedup 1.0000x reference)
import jax
import jax.numpy as jnp
from jax.experimental import pallas as pl


def kernel(hidden_states, gate_w, w_gate, w_up, w_down, shared_gate_w, shared_up_w, shared_down_w, correction_bias):
    raise NotImplementedError("write your pallas kernel here")


# speedup vs baseline: 2.6958x; 2.6958x over previous
import functools

import jax
import jax.numpy as jnp
from jax.experimental import pallas as pl
from jax.experimental.pallas import tpu as pltpu

T = 2048
D = 1024
E = 8
I = 512
ROUTED_SCALING = 2.5

TT = 1024
NT = T // TT


def _mlp(x_bf, wg, wu, wd):
    g = jax.lax.dot_general(x_bf, wg.astype(jnp.bfloat16),
                            (((1,), (1,)), ((), ())),
                            preferred_element_type=jnp.float32)
    u = jax.lax.dot_general(x_bf, wu.astype(jnp.bfloat16),
                            (((1,), (1,)), ((), ())),
                            preferred_element_type=jnp.float32)
    h = (g * jax.nn.sigmoid(g) * u).astype(jnp.bfloat16)
    y = jax.lax.dot_general(h, wd.astype(jnp.bfloat16),
                            (((1,), (1,)), ((), ())),
                            preferred_element_type=jnp.float32)
    return y


def _moe_body(x_ref, gw_ref, bias_ref, wg_ref, wu_ref, wd_ref,
              sg_ref, su_ref, sd_ref, out_ref, acc_ref, w_ref):
    e = pl.program_id(1)

    @pl.when(e == 0)
    def _router():
        x = x_ref[...]
        logits = jax.lax.dot_general(
            x.astype(jnp.bfloat16), gw_ref[...].astype(jnp.bfloat16),
            (((1,), (1,)), ((), ())),
            preferred_element_type=jnp.float32)
        scores = jax.nn.sigmoid(logits)
        s_choice = scores + bias_ref[...]
        col = jax.lax.broadcasted_iota(jnp.int32, s_choice.shape, 1)
        m1 = jnp.max(s_choice, axis=1, keepdims=True)
        i1 = jnp.min(jnp.where(s_choice >= m1, col, E), axis=1, keepdims=True)
        sel1 = col == i1
        s_excl = jnp.where(sel1, -1e30, s_choice)
        m2 = jnp.max(s_excl, axis=1, keepdims=True)
        i2 = jnp.min(jnp.where(s_excl >= m2, col, E), axis=1, keepdims=True)
        sel2 = col == i2
        sel = sel1 | sel2
        wsum = jnp.sum(jnp.where(sel, scores, 0.0), axis=1, keepdims=True)
        wd_dense = jnp.where(sel, scores, 0.0) * (ROUTED_SCALING / wsum)
        w_ref[...] = jnp.pad(wd_dense, ((0, 0), (0, 8)))
        acc_ref[...] = jnp.zeros_like(acc_ref)

    x_bf = x_ref[...].astype(jnp.bfloat16)

    @pl.when(e < 8)
    def _routed():
        y = _mlp(x_bf, wg_ref[0], wu_ref[0], wd_ref[0])
        lane = jax.lax.broadcasted_iota(jnp.int32, (TT, 16), 1)
        wcol = jnp.sum(jnp.where(lane == e, w_ref[...], 0.0), axis=1,
                       keepdims=True)
        acc_ref[...] += y * wcol

    @pl.when(e == 8)
    def _shared():
        y = _mlp(x_bf, sg_ref[...], su_ref[...], sd_ref[...])
        out_ref[...] = acc_ref[...] + y


@jax.jit
def kernel(hidden_states, gate_w, w_gate, w_up, w_down,
           shared_gate_w, shared_up_w, shared_down_w, correction_bias):
    bias = correction_bias.reshape(1, E).astype(jnp.float32)
    grid = (NT, E + 1)
    return pl.pallas_call(
        _moe_body,
        grid=grid,
        in_specs=[
            pl.BlockSpec((TT, D), lambda t, e: (t, 0)),
            pl.BlockSpec((E, D), lambda t, e: (0, 0)),
            pl.BlockSpec((1, E), lambda t, e: (0, 0)),
            pl.BlockSpec((1, I, D), lambda t, e: (jnp.minimum(e, 7), 0, 0)),
            pl.BlockSpec((1, I, D), lambda t, e: (jnp.minimum(e, 7), 0, 0)),
            pl.BlockSpec((1, D, I), lambda t, e: (jnp.minimum(e, 7), 0, 0)),
            pl.BlockSpec((I, D), lambda t, e: (0, 0)),
            pl.BlockSpec((I, D), lambda t, e: (0, 0)),
            pl.BlockSpec((D, I), lambda t, e: (0, 0)),
        ],
        out_specs=pl.BlockSpec((TT, D), lambda t, e: (t, 0)),
        out_shape=jax.ShapeDtypeStruct((T, D), jnp.float32),
        scratch_shapes=[
            pltpu.VMEM((TT, D), jnp.float32),
            pltpu.VMEM((TT, 16), jnp.float32),
        ],
    )(hidden_states, gate_w, bias, w_gate, w_up, w_down,
      shared_gate_w, shared_up_w, shared_down_w)
